# initial kernel scaffold (unmeasured)
import jax
import jax.numpy as jnp
from jax import lax
from jax.experimental import pallas as pl
from jax.experimental.pallas import tpu as pltpu

N_DEV = 32
LOG = 5
LAYERS = 3
SLOTS = LAYERS * LOG


def kernel(x, Win0, Wout0, Win1, Wout1, Win2, Wout2):
    b, d = x.shape

    def body(x_ref, win0, wout0, win1, wout1, win2, wout2,
             out_ref, acc_ref, recv_buf, send_sems, recv_sems):
        my = lax.axis_index("i")

        barrier = pltpu.get_barrier_semaphore()
        for k in range(LOG):
            partner = my ^ (1 << k)
            pl.semaphore_signal(
                barrier, inc=1,
                device_id=(partner,), device_id_type=pl.DeviceIdType.MESH,
            )
        pl.semaphore_wait(barrier, LOG)

        wins = [win0, win1, win2]
        wouts = [wout0, wout1, wout2]

        x_cur = x_ref[...]
        for layer in range(LAYERS):
            h = jnp.maximum(
                jnp.dot(x_cur, wins[layer][...],
                        preferred_element_type=jnp.float32),
                0.0,
            )
            acc_ref[...] = jnp.dot(h, wouts[layer][...],
                                   preferred_element_type=jnp.float32)
            for k in range(LOG):
                slot = layer * LOG + k
                partner = my ^ (1 << k)
                rdma = pltpu.make_async_remote_copy(
                    src_ref=acc_ref,
                    dst_ref=recv_buf.at[slot],
                    send_sem=send_sems.at[slot],
                    recv_sem=recv_sems.at[slot],
                    device_id=(partner,),
                    device_id_type=pl.DeviceIdType.MESH,
                )
                rdma.start()
                rdma.wait()
                acc_ref[...] = acc_ref[...] + recv_buf[slot]
            x_cur = acc_ref[...]

        rows = b // N_DEV
        out_ref[...] = acc_ref[pl.ds(my * rows, rows), :]

    return pl.pallas_call(
        body,
        out_shape=jax.ShapeDtypeStruct((b // N_DEV, d), jnp.float32),
        in_specs=[pl.BlockSpec(memory_space=pltpu.VMEM)] * 7,
        out_specs=pl.BlockSpec(memory_space=pltpu.VMEM),
        scratch_shapes=[
            pltpu.VMEM((b, d), jnp.float32),
            pltpu.VMEM((SLOTS, b, d), jnp.float32),
            pltpu.SemaphoreType.DMA((SLOTS,)),
            pltpu.SemaphoreType.DMA((SLOTS,)),
        ],
        compiler_params=pltpu.CompilerParams(collective_id=0),
    )(x, Win0, Wout0, Win1, Wout1, Win2, Wout2)


# baseline (device time: 45251 ns/iter reference)
import jax
import jax.numpy as jnp
from jax import lax
from jax.experimental import pallas as pl
from jax.experimental.pallas import tpu as pltpu

N_DEV = 32
MASKS = (1, 3, 4, 8, 16)
LOG = len(MASKS)
AR_LAYERS = 2
SLOTS = AR_LAYERS * LOG


def kernel(x, Win0, Wout0, Win1, Wout1, Win2, Wout2):
    b, d = x.shape
    rows = b // N_DEV

    def body(x_ref, win0, wout0, win1, wout1, win2, wout2,
             out_ref, acc_ref, recv_buf, send_sems, recv_sems,
             recv2, send_sems2, recv_sems2):
        my = lax.axis_index("i")

        barrier = pltpu.get_barrier_semaphore()
        for m in MASKS:
            pl.semaphore_signal(
                barrier, inc=1,
                device_id=(my ^ m,), device_id_type=pl.DeviceIdType.MESH,
            )

        wins = [win0, win1, win2]
        wouts = [wout0, wout1, wout2]

        def layer_partial(x_val, layer):
            h = jnp.maximum(
                jnp.dot(x_val, wins[layer][...],
                        preferred_element_type=jnp.float32),
                0.0,
            )
            return jnp.dot(h, wouts[layer][...],
                           preferred_element_type=jnp.float32)

        acc_ref[...] = layer_partial(x_ref[...], 0)
        pl.semaphore_wait(barrier, LOG)

        for layer in range(AR_LAYERS):
            for k, m in enumerate(MASKS):
                slot = layer * LOG + k
                rdma = pltpu.make_async_remote_copy(
                    src_ref=acc_ref,
                    dst_ref=recv_buf.at[slot],
                    send_sem=send_sems.at[slot],
                    recv_sem=recv_sems.at[slot],
                    device_id=(my ^ m,),
                    device_id_type=pl.DeviceIdType.MESH,
                )
                rdma.start()
                rdma.wait()
                acc_ref[...] = acc_ref[...] + recv_buf[slot]
            acc_ref[...] = layer_partial(acc_ref[...], layer + 1)

        sends = []
        for j in range(N_DEV):
            out_rdma = pltpu.make_async_remote_copy(
                src_ref=acc_ref.at[pl.ds(j * rows, rows), :],
                dst_ref=recv2.at[pl.ds(my * rows, rows), :],
                send_sem=send_sems2.at[j],
                recv_sem=recv_sems2.at[my],
                device_id=(j,),
                device_id_type=pl.DeviceIdType.MESH,
            )
            sends.append(out_rdma)

            @pl.when(my != j)
            def _(out_rdma=out_rdma):
                out_rdma.start()

        recv2[pl.ds(my * rows, rows), :] = acc_ref[pl.ds(my * rows, rows), :]

        for s in range(N_DEV):
            in_rdma = pltpu.make_async_remote_copy(
                src_ref=acc_ref.at[pl.ds(0, rows), :],
                dst_ref=recv2.at[pl.ds(s * rows, rows), :],
                send_sem=send_sems2.at[s],
                recv_sem=recv_sems2.at[s],
                device_id=(s,),
                device_id_type=pl.DeviceIdType.MESH,
            )

            @pl.when(my != s)
            def _(in_rdma=in_rdma):
                in_rdma.wait_recv()

        out_ref[...] = jnp.sum(
            recv2[...].reshape(N_DEV, rows, d), axis=0
        )

        for j in range(N_DEV):
            @pl.when(my != j)
            def _(out_rdma=sends[j]):
                out_rdma.wait_send()

    return pl.pallas_call(
        body,
        out_shape=jax.ShapeDtypeStruct((rows, d), jnp.float32),
        in_specs=[pl.BlockSpec(memory_space=pltpu.VMEM)] * 7,
        out_specs=pl.BlockSpec(memory_space=pltpu.VMEM),
        scratch_shapes=[
            pltpu.VMEM((b, d), jnp.float32),
            pltpu.VMEM((SLOTS, b, d), jnp.float32),
            pltpu.SemaphoreType.DMA((SLOTS,)),
            pltpu.SemaphoreType.DMA((SLOTS,)),
            pltpu.VMEM((b, d), jnp.float32),
            pltpu.SemaphoreType.DMA((N_DEV,)),
            pltpu.SemaphoreType.DMA((N_DEV,)),
        ],
        compiler_params=pltpu.CompilerParams(collective_id=0),
    )(x, Win0, Wout0, Win1, Wout1, Win2, Wout2)


# device time: 39994 ns/iter; 1.1314x vs baseline; 1.1314x over previous
import os

import jax
import jax.numpy as jnp
from jax import lax
from jax.experimental import pallas as pl
from jax.experimental.pallas import tpu as pltpu

PROBE = int(os.environ.get("PROBE", "0"))
SKIP_BUTTERFLY = PROBE in (1, 3, 4, 5)
SKIP_SCATTER = PROBE in (2, 3, 4, 5)
SKIP_BARRIER = PROBE in (1, 3, 4, 5)
SKIP_COMPUTE = PROBE in (4, 5)
NO_STAGE = PROBE == 5

N_DEV = 32
PHASES = ((1,), (7, 4, 3), (24, 16, 8))
MASKS = tuple(m for ph in PHASES for m in ph)
LOG = len(MASKS)
AR_LAYERS = 2
SLOTS = AR_LAYERS * LOG


def kernel(x, Win0, Wout0, Win1, Wout1, Win2, Wout2):
    b, d = x.shape
    rows = b // N_DEV

    def body(x_ref, win0, wout0, win1, wout1, win2, wout2,
             out_ref, acc_a, acc_b, recv_buf, send_sems, recv_sems,
             recv2, send_sems2, recv_sems2):
        my = lax.axis_index("i")
        if NO_STAGE:
            out_ref[...] = jnp.zeros_like(out_ref)
            return

        BARRIER_W = {1: 28, 7: 8, 4: 8, 3: 8, 24: 1, 16: 1, 8: 1}
        STAGE_WAITS = (28, 24, 3)
        if not SKIP_BARRIER:
            barrier = pltpu.get_barrier_semaphore()
            for m, w in BARRIER_W.items():
                pl.semaphore_signal(
                    barrier, inc=w,
                    device_id=(my ^ m,), device_id_type=pl.DeviceIdType.MESH,
                )

        wins = [win0, win1, win2]
        wouts = [wout0, wout1, wout2]

        def layer_partial(x_val, layer):
            if SKIP_COMPUTE:
                return x_val
            h = jnp.maximum(
                jnp.dot(x_val, wins[layer][...],
                        preferred_element_type=jnp.float32),
                0.0,
            )
            return jnp.dot(h, wouts[layer][...],
                           preferred_element_type=jnp.float32)

        accs = [acc_a, acc_b]
        accs[0][...] = layer_partial(x_ref[...], 0)

        cur = 0
        pending = []
        for layer in range(AR_LAYERS):
            k = 0
            for p_i, phase in enumerate(PHASES if not SKIP_BUTTERFLY else ()):
                if layer == 0 and not SKIP_BARRIER:
                    pl.semaphore_wait(barrier, STAGE_WAITS[p_i])
                rds = []
                for m in phase:
                    slot = layer * LOG + k
                    k += 1
                    rdma = pltpu.make_async_remote_copy(
                        src_ref=accs[cur],
                        dst_ref=recv_buf.at[slot],
                        send_sem=send_sems.at[slot],
                        recv_sem=recv_sems.at[slot],
                        device_id=(my ^ m,),
                        device_id_type=pl.DeviceIdType.MESH,
                    )
                    rdma.start()
                    rds.append((rdma, slot))
                for rdma, _ in rds:
                    rdma.wait_recv()
                for p in pending:
                    p.wait_send()
                total = accs[cur][...]
                for _, slot in rds:
                    total = total + recv_buf[slot]
                accs[1 - cur][...] = total
                pending = [rdma for rdma, _ in rds]
                cur = 1 - cur
            for p in pending:
                p.wait_send()
            pending = []
            accs[1 - cur][...] = layer_partial(accs[cur][...], layer + 1)
            cur = 1 - cur
        acc_ref = accs[cur]

        sends = []
        for j in range(0 if SKIP_SCATTER else N_DEV):
            out_rdma = pltpu.make_async_remote_copy(
                src_ref=acc_ref.at[pl.ds(j * rows, rows), :],
                dst_ref=recv2.at[pl.ds(my * rows, rows), :],
                send_sem=send_sems2.at[j],
                recv_sem=recv_sems2.at[my],
                device_id=(j,),
                device_id_type=pl.DeviceIdType.MESH,
            )
            sends.append(out_rdma)

            @pl.when(my != j)
            def _(out_rdma=out_rdma):
                out_rdma.start()

        recv2[pl.ds(my * rows, rows), :] = acc_ref[pl.ds(my * rows, rows), :]

        for s in range(0 if SKIP_SCATTER else N_DEV):
            in_rdma = pltpu.make_async_remote_copy(
                src_ref=acc_ref.at[pl.ds(0, rows), :],
                dst_ref=recv2.at[pl.ds(s * rows, rows), :],
                send_sem=send_sems2.at[s],
                recv_sem=recv_sems2.at[s],
                device_id=(s,),
                device_id_type=pl.DeviceIdType.MESH,
            )

            @pl.when(my != s)
            def _(in_rdma=in_rdma):
                in_rdma.wait_recv()

        out_ref[...] = jnp.sum(
            recv2[...].reshape(N_DEV, rows, d), axis=0
        )

        for j in range(0 if SKIP_SCATTER else N_DEV):
            @pl.when(my != j)
            def _(out_rdma=sends[j]):
                out_rdma.wait_send()

    return pl.pallas_call(
        body,
        out_shape=jax.ShapeDtypeStruct((rows, d), jnp.float32),
        in_specs=[pl.BlockSpec(
            memory_space=pl.ANY if NO_STAGE else pltpu.VMEM)] * 7,
        out_specs=pl.BlockSpec(memory_space=pltpu.VMEM),
        scratch_shapes=[
            pltpu.VMEM((b, d), jnp.float32),
            pltpu.VMEM((b, d), jnp.float32),
            pltpu.VMEM((SLOTS, b, d), jnp.float32),
            pltpu.SemaphoreType.DMA((SLOTS,)),
            pltpu.SemaphoreType.DMA((SLOTS,)),
            pltpu.VMEM((b, d), jnp.float32),
            pltpu.SemaphoreType.DMA((N_DEV,)),
            pltpu.SemaphoreType.DMA((N_DEV,)),
        ],
        compiler_params=(
            None if SKIP_BARRIER
            else pltpu.CompilerParams(collective_id=0)
        ),
    )(x, Win0, Wout0, Win1, Wout1, Win2, Wout2)
